# Initial kernel scaffold; baseline (speedup 1.0000x reference)
import jax
import jax.numpy as jnp
from jax.experimental import pallas as pl


def kernel(input_lr, embed):
    raise NotImplementedError("write your pallas kernel here")


# speedup vs baseline: 1.9923x; 1.9923x over previous
import jax
import jax.numpy as jnp
from jax.experimental import pallas as pl
from jax.experimental.pallas import tpu as pltpu

DIM_ = 256
NE_ = 8192
TM_ = 256


def _dist_body(x_ref, e_ref, d_ref, ind_ref):
    xb = x_ref[...]
    s = jnp.sum(xb * xb, axis=1, keepdims=True)
    e = e_ref[...]
    e2 = jnp.sum(e * e, axis=0, keepdims=True)
    m = jnp.dot(xb.astype(jnp.bfloat16), e.astype(jnp.bfloat16),
                preferred_element_type=jnp.float32)
    d = (s - 2.0 * m) + e2
    d_ref[...] = d
    minv = jnp.min(d, axis=1, keepdims=True)
    iota = jax.lax.broadcasted_iota(jnp.int32, d.shape, 1)
    ind = jnp.min(jnp.where(d == minv, iota, jnp.int32(2**30)), axis=1)
    ind_ref[0, 0, :] = ind


def _fixup_body(x_ref, g_ref, q_ref, diff_ref, acc_ref):
    i = pl.program_id(0)
    xb = x_ref[...]
    gb = g_ref[...]
    t = gb - xb
    q_ref[...] = xb + t
    blocksum = jnp.sum(t * t)

    @pl.when(i == 0)
    def _():
        acc_ref[0] = 0.0

    acc_ref[0] += blocksum

    @pl.when(i == pl.num_programs(0) - 1)
    def _():
        diff_ref[...] = jnp.full((1, 1), acc_ref[0] / (16384.0 * 256.0),
                                 jnp.float32)


def kernel(input_lr, embed):
    n = input_lr.shape[0] * input_lr.shape[1] * input_lr.shape[2]
    x = input_lr.reshape(n, DIM_)
    nrt = n // TM_

    dist, ind3 = pl.pallas_call(
        _dist_body,
        grid=(nrt,),
        in_specs=[
            pl.BlockSpec((TM_, DIM_), lambda i: (i, 0)),
            pl.BlockSpec((DIM_, NE_), lambda i: (0, 0)),
        ],
        out_specs=[
            pl.BlockSpec((TM_, NE_), lambda i: (i, 0)),
            pl.BlockSpec((1, 1, TM_), lambda i: (i, 0, 0)),
        ],
        out_shape=[
            jax.ShapeDtypeStruct((n, NE_), jnp.float32),
            jax.ShapeDtypeStruct((nrt, 1, TM_), jnp.int32),
        ],
    )(x, embed)

    ind_flat = ind3.reshape(n)
    g = jnp.take(embed.T, ind_flat, axis=0)

    q2, diff2 = pl.pallas_call(
        _fixup_body,
        grid=(nrt,),
        in_specs=[
            pl.BlockSpec((TM_, DIM_), lambda i: (i, 0)),
            pl.BlockSpec((TM_, DIM_), lambda i: (i, 0)),
        ],
        out_specs=[
            pl.BlockSpec((TM_, DIM_), lambda i: (i, 0)),
            pl.BlockSpec((1, 1), lambda i: (0, 0)),
        ],
        out_shape=[
            jax.ShapeDtypeStruct((n, DIM_), jnp.float32),
            jax.ShapeDtypeStruct((1, 1), jnp.float32),
        ],
        scratch_shapes=[pltpu.SMEM((1,), jnp.float32)],
    )(x, g)

    quantize = q2.reshape(input_lr.shape)
    diff = diff2.reshape(())
    embed_ind = ind_flat.reshape(input_lr.shape[:-1])
    return (quantize, diff, embed_ind, dist)


# speedup vs baseline: 2.2909x; 1.1498x over previous
import functools

import jax
import jax.numpy as jnp
from jax import lax
from jax.experimental import pallas as pl
from jax.experimental.pallas import tpu as pltpu
from jax.experimental.pallas import tpu_sc as plsc

DIM_ = 256
NE_ = 8192
TM_ = 256
N_ = 16384
SC_WORKERS_ = 32
SC_CHUNK_ = 256


def _dist_body(x_ref, e_ref, d_ref, ind_ref):
    xb = x_ref[...]
    s = jnp.sum(xb * xb, axis=1, keepdims=True)
    e = e_ref[...]
    e2 = jnp.sum(e * e, axis=0, keepdims=True)
    m = jnp.dot(xb.astype(jnp.bfloat16), e.astype(jnp.bfloat16),
                preferred_element_type=jnp.float32)
    d = (s - 2.0 * m) + e2
    d_ref[...] = d
    minv = jnp.min(d, axis=1, keepdims=True)
    iota = jax.lax.broadcasted_iota(jnp.int32, d.shape, 1)
    ind = jnp.min(jnp.where(d == minv, iota, jnp.int32(2**30)), axis=1)
    ind_ref[0, 0, :] = ind


def _transpose_body(e_ref, et_ref):
    et_ref[...] = e_ref[...].T


def _sc_gather(embed_t, ind_flat):
    mesh = plsc.VectorSubcoreMesh(core_axis_name="c", subcore_axis_name="s")
    b_per_w = N_ // SC_WORKERS_

    @functools.partial(
        pl.kernel, mesh=mesh,
        out_type=jax.ShapeDtypeStruct((N_, DIM_), jnp.float32),
        scratch_types=[
            pltpu.VMEM((SC_CHUNK_,), jnp.int32),
            pltpu.VMEM((SC_CHUNK_, DIM_), jnp.float32),
            pltpu.SemaphoreType.DMA,
        ],
    )
    def k(table_hbm, idx_hbm, out_hbm, idx_v, rows_v, sem):
        wid = lax.axis_index("s") * 2 + lax.axis_index("c")
        base = wid * b_per_w

        @pl.loop(0, b_per_w, step=SC_CHUNK_)
        def _(off):
            pltpu.sync_copy(idx_hbm.at[pl.ds(base + off, SC_CHUNK_)], idx_v)
            pltpu.async_copy(table_hbm.at[idx_v], rows_v, sem).wait()
            pltpu.sync_copy(rows_v, out_hbm.at[pl.ds(base + off, SC_CHUNK_)])

    return k(embed_t, ind_flat)


def _fixup_body(x_ref, g_ref, q_ref, diff_ref, acc_ref):
    i = pl.program_id(0)
    xb = x_ref[...]
    gb = g_ref[...]
    t = gb - xb
    q_ref[...] = xb + t
    blocksum = jnp.sum(t * t)

    @pl.when(i == 0)
    def _():
        acc_ref[0] = 0.0

    acc_ref[0] += blocksum

    @pl.when(i == pl.num_programs(0) - 1)
    def _():
        diff_ref[...] = jnp.full((1, 1), acc_ref[0] / (16384.0 * 256.0),
                                 jnp.float32)


def kernel(input_lr, embed):
    n = input_lr.shape[0] * input_lr.shape[1] * input_lr.shape[2]
    x = input_lr.reshape(n, DIM_)
    nrt = n // TM_

    dist, ind3 = pl.pallas_call(
        _dist_body,
        grid=(nrt,),
        in_specs=[
            pl.BlockSpec((TM_, DIM_), lambda i: (i, 0)),
            pl.BlockSpec((DIM_, NE_), lambda i: (0, 0)),
        ],
        out_specs=[
            pl.BlockSpec((TM_, NE_), lambda i: (i, 0)),
            pl.BlockSpec((1, 1, TM_), lambda i: (i, 0, 0)),
        ],
        out_shape=[
            jax.ShapeDtypeStruct((n, NE_), jnp.float32),
            jax.ShapeDtypeStruct((nrt, 1, TM_), jnp.int32),
        ],
    )(x, embed)

    ind_flat = ind3.reshape(n)

    embed_t = pl.pallas_call(
        _transpose_body,
        grid=(16,),
        in_specs=[pl.BlockSpec((DIM_, NE_ // 16), lambda j: (0, j))],
        out_specs=pl.BlockSpec((NE_ // 16, DIM_), lambda j: (j, 0)),
        out_shape=jax.ShapeDtypeStruct((NE_, DIM_), jnp.float32),
    )(embed)

    g = _sc_gather(embed_t, ind_flat)

    q2, diff2 = pl.pallas_call(
        _fixup_body,
        grid=(nrt,),
        in_specs=[
            pl.BlockSpec((TM_, DIM_), lambda i: (i, 0)),
            pl.BlockSpec((TM_, DIM_), lambda i: (i, 0)),
        ],
        out_specs=[
            pl.BlockSpec((TM_, DIM_), lambda i: (i, 0)),
            pl.BlockSpec((1, 1), lambda i: (0, 0)),
        ],
        out_shape=[
            jax.ShapeDtypeStruct((n, DIM_), jnp.float32),
            jax.ShapeDtypeStruct((1, 1), jnp.float32),
        ],
        scratch_shapes=[pltpu.SMEM((1,), jnp.float32)],
    )(x, g)

    quantize = q2.reshape(input_lr.shape)
    diff = diff2.reshape(())
    embed_ind = ind_flat.reshape(input_lr.shape[:-1])
    return (quantize, diff, embed_ind, dist)


# speedup vs baseline: 2.8292x; 1.2350x over previous
import functools

import jax
import jax.numpy as jnp
from jax import lax
from jax.experimental import pallas as pl
from jax.experimental.pallas import tpu as pltpu
from jax.experimental.pallas import tpu_sc as plsc

DIM_ = 256
NE_ = 8192
TM_ = 256
N_ = 16384
SC_WORKERS_ = 32
SC_CHUNK_ = 256


W_ = 128


def _dist_body(x_ref, eb_ref, e2_ref, d_ref, ind_ref):
    xb = x_ref[...]
    s = jnp.sum(xb * xb, axis=1, keepdims=True)
    e2 = e2_ref[...]
    m = jnp.dot(xb.astype(jnp.bfloat16), eb_ref[...],
                preferred_element_type=jnp.float32)
    d = (s - 2.0 * m) + e2
    d_ref[...] = d
    nch = NE_ // W_
    rv = d[:, :W_]
    for j in range(1, nch):
        rv = jnp.minimum(rv, d[:, j * W_:(j + 1) * W_])
    minv = jnp.min(rv, axis=1, keepdims=True)
    big = jnp.int32(2**30)
    ri = jnp.where(d[:, :W_] == minv, jnp.int32(0), big)
    for j in range(1, nch):
        cand = jnp.where(d[:, j * W_:(j + 1) * W_] == minv, jnp.int32(j), big)
        ri = jnp.minimum(ri, cand)
    lane = jax.lax.broadcasted_iota(jnp.int32, (TM_, W_), 1)
    idx_full = jnp.where(ri == big, big, ri * W_ + lane)
    ind = jnp.min(idx_full, axis=1)
    ind_ref[0, 0, :] = ind


def _prep_body(e_ref, et_ref, eb_ref, e2_ref):
    e = e_ref[...]
    et_ref[...] = e.T
    eb_ref[...] = e.astype(jnp.bfloat16)
    e2_ref[...] = jnp.sum(e * e, axis=0, keepdims=True)


def _sc_gather(embed_t, ind_flat):
    mesh = plsc.VectorSubcoreMesh(core_axis_name="c", subcore_axis_name="s")
    b_per_w = N_ // SC_WORKERS_

    @functools.partial(
        pl.kernel, mesh=mesh,
        out_type=jax.ShapeDtypeStruct((N_, DIM_), jnp.float32),
        scratch_types=[
            pltpu.VMEM((SC_CHUNK_,), jnp.int32),
            pltpu.VMEM((SC_CHUNK_, DIM_), jnp.float32),
            pltpu.SemaphoreType.DMA,
        ],
    )
    def k(table_hbm, idx_hbm, out_hbm, idx_v, rows_v, sem):
        wid = lax.axis_index("s") * 2 + lax.axis_index("c")
        base = wid * b_per_w

        @pl.loop(0, b_per_w, step=SC_CHUNK_)
        def _(off):
            pltpu.sync_copy(idx_hbm.at[pl.ds(base + off, SC_CHUNK_)], idx_v)
            pltpu.async_copy(table_hbm.at[idx_v], rows_v, sem).wait()
            pltpu.sync_copy(rows_v, out_hbm.at[pl.ds(base + off, SC_CHUNK_)])

    return k(embed_t, ind_flat)


def _fixup_body(x_ref, g_ref, q_ref, diff_ref, acc_ref):
    i = pl.program_id(0)
    xb = x_ref[...]
    gb = g_ref[...]
    t = gb - xb
    q_ref[...] = xb + t
    blocksum = jnp.sum(t * t)

    @pl.when(i == 0)
    def _():
        acc_ref[0] = 0.0

    acc_ref[0] += blocksum

    @pl.when(i == pl.num_programs(0) - 1)
    def _():
        diff_ref[...] = jnp.full((1, 1), acc_ref[0] / (16384.0 * 256.0),
                                 jnp.float32)


def kernel(input_lr, embed):
    n = input_lr.shape[0] * input_lr.shape[1] * input_lr.shape[2]
    x = input_lr.reshape(n, DIM_)
    nrt = n // TM_

    embed_t, embed_b16, e2 = pl.pallas_call(
        _prep_body,
        grid=(16,),
        in_specs=[pl.BlockSpec((DIM_, NE_ // 16), lambda j: (0, j))],
        out_specs=[
            pl.BlockSpec((NE_ // 16, DIM_), lambda j: (j, 0)),
            pl.BlockSpec((DIM_, NE_ // 16), lambda j: (0, j)),
            pl.BlockSpec((1, NE_ // 16), lambda j: (0, j)),
        ],
        out_shape=[
            jax.ShapeDtypeStruct((NE_, DIM_), jnp.float32),
            jax.ShapeDtypeStruct((DIM_, NE_), jnp.bfloat16),
            jax.ShapeDtypeStruct((1, NE_), jnp.float32),
        ],
    )(embed)

    dist, ind3 = pl.pallas_call(
        _dist_body,
        grid=(nrt,),
        in_specs=[
            pl.BlockSpec((TM_, DIM_), lambda i: (i, 0)),
            pl.BlockSpec((DIM_, NE_), lambda i: (0, 0)),
            pl.BlockSpec((1, NE_), lambda i: (0, 0)),
        ],
        out_specs=[
            pl.BlockSpec((TM_, NE_), lambda i: (i, 0)),
            pl.BlockSpec((1, 1, TM_), lambda i: (i, 0, 0)),
        ],
        out_shape=[
            jax.ShapeDtypeStruct((n, NE_), jnp.float32),
            jax.ShapeDtypeStruct((nrt, 1, TM_), jnp.int32),
        ],
    )(x, embed_b16, e2)

    ind_flat = ind3.reshape(n)

    g = _sc_gather(embed_t, ind_flat)

    q2, diff2 = pl.pallas_call(
        _fixup_body,
        grid=(nrt,),
        in_specs=[
            pl.BlockSpec((TM_, DIM_), lambda i: (i, 0)),
            pl.BlockSpec((TM_, DIM_), lambda i: (i, 0)),
        ],
        out_specs=[
            pl.BlockSpec((TM_, DIM_), lambda i: (i, 0)),
            pl.BlockSpec((1, 1), lambda i: (0, 0)),
        ],
        out_shape=[
            jax.ShapeDtypeStruct((n, DIM_), jnp.float32),
            jax.ShapeDtypeStruct((1, 1), jnp.float32),
        ],
        scratch_shapes=[pltpu.SMEM((1,), jnp.float32)],
    )(x, g)

    quantize = q2.reshape(input_lr.shape)
    diff = diff2.reshape(())
    embed_ind = ind_flat.reshape(input_lr.shape[:-1])
    return (quantize, diff, embed_ind, dist)


# speedup vs baseline: 2.9473x; 1.0418x over previous
import functools

import jax
import jax.numpy as jnp
from jax import lax
from jax.experimental import pallas as pl
from jax.experimental.pallas import tpu as pltpu
from jax.experimental.pallas import tpu_sc as plsc

DIM_ = 256
NE_ = 8192
TM_ = 256
N_ = 16384
SC_WORKERS_ = 32
SC_CHUNK_ = 256


W_ = 128


def _dist_body(x_ref, eb_ref, e2_ref, d_ref, ind_ref):
    xb = x_ref[...]
    s = jnp.sum(xb * xb, axis=1, keepdims=True)
    e2 = e2_ref[...]
    m2 = jnp.dot(xb.astype(jnp.bfloat16), eb_ref[...],
                 preferred_element_type=jnp.float32)
    nch = NE_ // W_
    RH = TM_ // 4
    for h in range(TM_ // RH):
        rows = slice(h * RH, (h + 1) * RH)
        sh = s[rows]
        mh = m2[rows]
        rv = ri = None
        for j in range(nch):
            cols = slice(j * W_, (j + 1) * W_)
            dj = (sh + mh[:, cols]) + e2[:, cols]
            d_ref[rows, cols] = dj
            if j == 0:
                rv = dj
                ri = jnp.zeros((RH, W_), jnp.int32)
            else:
                upd = dj < rv
                rv = jnp.where(upd, dj, rv)
                ri = jnp.where(upd, jnp.int32(j), ri)
        minv = jnp.min(rv, axis=1, keepdims=True)
        lane = jax.lax.broadcasted_iota(jnp.int32, (RH, W_), 1)
        big = jnp.int32(2**30)
        idx_full = jnp.where(rv == minv, ri * W_ + lane, big)
        ind = jnp.min(idx_full, axis=1)
        ind_ref[0, 0, rows] = ind


def _prep_body(e_ref, et_ref, eb_ref, e2_ref):
    e = e_ref[...]
    et_ref[...] = e.T
    eb_ref[...] = (e * -2.0).astype(jnp.bfloat16)
    e2_ref[...] = jnp.sum(e * e, axis=0, keepdims=True)


def _sc_gather(embed_t, ind_flat):
    mesh = plsc.VectorSubcoreMesh(core_axis_name="c", subcore_axis_name="s")
    b_per_w = N_ // SC_WORKERS_

    @functools.partial(
        pl.kernel, mesh=mesh,
        out_type=jax.ShapeDtypeStruct((N_, DIM_), jnp.float32),
        scratch_types=[
            pltpu.VMEM((SC_CHUNK_,), jnp.int32),
            pltpu.VMEM((SC_CHUNK_, DIM_), jnp.float32),
            pltpu.SemaphoreType.DMA,
        ],
    )
    def k(table_hbm, idx_hbm, out_hbm, idx_v, rows_v, sem):
        wid = lax.axis_index("s") * 2 + lax.axis_index("c")
        base = wid * b_per_w

        @pl.loop(0, b_per_w, step=SC_CHUNK_)
        def _(off):
            pltpu.sync_copy(idx_hbm.at[pl.ds(base + off, SC_CHUNK_)], idx_v)
            pltpu.async_copy(table_hbm.at[idx_v], rows_v, sem).wait()
            pltpu.sync_copy(rows_v, out_hbm.at[pl.ds(base + off, SC_CHUNK_)])

    return k(embed_t, ind_flat)


def _fixup_body(x_ref, g_ref, q_ref, diff_ref, acc_ref):
    i = pl.program_id(0)
    xb = x_ref[...]
    gb = g_ref[...]
    t = gb - xb
    q_ref[...] = xb + t
    blocksum = jnp.sum(t * t)

    @pl.when(i == 0)
    def _():
        acc_ref[0] = 0.0

    acc_ref[0] += blocksum

    @pl.when(i == pl.num_programs(0) - 1)
    def _():
        diff_ref[...] = jnp.full((1, 1), acc_ref[0] / (16384.0 * 256.0),
                                 jnp.float32)


def kernel(input_lr, embed):
    n = input_lr.shape[0] * input_lr.shape[1] * input_lr.shape[2]
    x = input_lr.reshape(n, DIM_)
    nrt = n // TM_

    embed_t, embed_b16, e2 = pl.pallas_call(
        _prep_body,
        grid=(16,),
        in_specs=[pl.BlockSpec((DIM_, NE_ // 16), lambda j: (0, j))],
        out_specs=[
            pl.BlockSpec((NE_ // 16, DIM_), lambda j: (j, 0)),
            pl.BlockSpec((DIM_, NE_ // 16), lambda j: (0, j)),
            pl.BlockSpec((1, NE_ // 16), lambda j: (0, j)),
        ],
        out_shape=[
            jax.ShapeDtypeStruct((NE_, DIM_), jnp.float32),
            jax.ShapeDtypeStruct((DIM_, NE_), jnp.bfloat16),
            jax.ShapeDtypeStruct((1, NE_), jnp.float32),
        ],
    )(embed)

    dist, ind3 = pl.pallas_call(
        _dist_body,
        grid=(nrt,),
        in_specs=[
            pl.BlockSpec((TM_, DIM_), lambda i: (i, 0)),
            pl.BlockSpec((DIM_, NE_), lambda i: (0, 0)),
            pl.BlockSpec((1, NE_), lambda i: (0, 0)),
        ],
        out_specs=[
            pl.BlockSpec((TM_, NE_), lambda i: (i, 0)),
            pl.BlockSpec((1, 1, TM_), lambda i: (i, 0, 0)),
        ],
        out_shape=[
            jax.ShapeDtypeStruct((n, NE_), jnp.float32),
            jax.ShapeDtypeStruct((nrt, 1, TM_), jnp.int32),
        ],
    )(x, embed_b16, e2)

    ind_flat = ind3.reshape(n)

    g = _sc_gather(embed_t, ind_flat)

    q2, diff2 = pl.pallas_call(
        _fixup_body,
        grid=(nrt,),
        in_specs=[
            pl.BlockSpec((TM_, DIM_), lambda i: (i, 0)),
            pl.BlockSpec((TM_, DIM_), lambda i: (i, 0)),
        ],
        out_specs=[
            pl.BlockSpec((TM_, DIM_), lambda i: (i, 0)),
            pl.BlockSpec((1, 1), lambda i: (0, 0)),
        ],
        out_shape=[
            jax.ShapeDtypeStruct((n, DIM_), jnp.float32),
            jax.ShapeDtypeStruct((1, 1), jnp.float32),
        ],
        scratch_shapes=[pltpu.SMEM((1,), jnp.float32)],
    )(x, g)

    quantize = q2.reshape(input_lr.shape)
    diff = diff2.reshape(())
    embed_ind = ind_flat.reshape(input_lr.shape[:-1])
    return (quantize, diff, embed_ind, dist)


# speedup vs baseline: 3.4817x; 1.1813x over previous
import functools

import jax
import jax.numpy as jnp
from jax import lax
from jax.experimental import pallas as pl
from jax.experimental.pallas import tpu as pltpu
from jax.experimental.pallas import tpu_sc as plsc

DIM_ = 256
NE_ = 8192
TM_ = 256
N_ = 16384
SC_WORKERS_ = 32
SC_CHUNK_ = 256


W_ = 128


def _dist_body(x_ref, e_ref, d_ref, ind_ref, et_ref, diff_ref,
               eb_ref, e2_ref, acc_ref):
    i = pl.program_id(0)

    @pl.when(i == 0)
    def _prep():
        e = e_ref[...]
        et_ref[...] = e.T
        eb_ref[...] = (e * -2.0).astype(jnp.bfloat16)
        e2_ref[...] = jnp.sum(e * e, axis=0, keepdims=True)
        acc_ref[0] = 0.0

    xb = x_ref[...]
    s = jnp.sum(xb * xb, axis=1, keepdims=True)
    e2 = e2_ref[...]
    m2 = jnp.dot(xb.astype(jnp.bfloat16), eb_ref[...],
                 preferred_element_type=jnp.float32)
    nch = NE_ // W_
    RH = TM_ // 4
    for h in range(TM_ // RH):
        rows = slice(h * RH, (h + 1) * RH)
        sh = s[rows]
        mh = m2[rows]
        rv = ri = None
        for j in range(nch):
            cols = slice(j * W_, (j + 1) * W_)
            dj = (sh + mh[:, cols]) + e2[:, cols]
            d_ref[rows, cols] = dj
            if j == 0:
                rv = dj
                ri = jnp.zeros((RH, W_), jnp.int32)
            else:
                upd = dj < rv
                rv = jnp.where(upd, dj, rv)
                ri = jnp.where(upd, jnp.int32(j), ri)
        minv = jnp.min(rv, axis=1, keepdims=True)
        lane = jax.lax.broadcasted_iota(jnp.int32, (RH, W_), 1)
        big = jnp.int32(2**30)
        idx_full = jnp.where(rv == minv, ri * W_ + lane, big)
        ind = jnp.min(idx_full, axis=1)
        ind_ref[0, 0, rows] = ind
        acc_ref[0] += jnp.sum(minv)

    @pl.when(i == pl.num_programs(0) - 1)
    def _fin():
        diff_ref[...] = jnp.full((1, 1), acc_ref[0] / (16384.0 * 256.0),
                                 jnp.float32)


def _sc_gather(embed_t, ind_flat):
    mesh = plsc.VectorSubcoreMesh(core_axis_name="c", subcore_axis_name="s")
    b_per_w = N_ // SC_WORKERS_

    @functools.partial(
        pl.kernel, mesh=mesh,
        out_type=jax.ShapeDtypeStruct((N_, DIM_), jnp.float32),
        scratch_types=[
            pltpu.VMEM((SC_CHUNK_,), jnp.int32),
            pltpu.VMEM((SC_CHUNK_, DIM_), jnp.float32),
            pltpu.SemaphoreType.DMA,
        ],
    )
    def k(table_hbm, idx_hbm, out_hbm, idx_v, rows_v, sem):
        wid = lax.axis_index("s") * 2 + lax.axis_index("c")
        base = wid * b_per_w

        @pl.loop(0, b_per_w, step=SC_CHUNK_)
        def _(off):
            pltpu.sync_copy(idx_hbm.at[pl.ds(base + off, SC_CHUNK_)], idx_v)
            pltpu.async_copy(table_hbm.at[idx_v], rows_v, sem).wait()
            pltpu.sync_copy(rows_v, out_hbm.at[pl.ds(base + off, SC_CHUNK_)])

    return k(embed_t, ind_flat)


def kernel(input_lr, embed):
    n = input_lr.shape[0] * input_lr.shape[1] * input_lr.shape[2]
    x = input_lr.reshape(n, DIM_)
    nrt = n // TM_

    dist, ind3, embed_t, diff2 = pl.pallas_call(
        _dist_body,
        grid=(nrt,),
        in_specs=[
            pl.BlockSpec((TM_, DIM_), lambda i: (i, 0)),
            pl.BlockSpec((DIM_, NE_), lambda i: (0, 0)),
        ],
        out_specs=[
            pl.BlockSpec((TM_, NE_), lambda i: (i, 0)),
            pl.BlockSpec((1, 1, TM_), lambda i: (i, 0, 0)),
            pl.BlockSpec((NE_, DIM_), lambda i: (0, 0)),
            pl.BlockSpec((1, 1), lambda i: (0, 0)),
        ],
        out_shape=[
            jax.ShapeDtypeStruct((n, NE_), jnp.float32),
            jax.ShapeDtypeStruct((nrt, 1, TM_), jnp.int32),
            jax.ShapeDtypeStruct((NE_, DIM_), jnp.float32),
            jax.ShapeDtypeStruct((1, 1), jnp.float32),
        ],
        scratch_shapes=[
            pltpu.VMEM((DIM_, NE_), jnp.bfloat16),
            pltpu.VMEM((1, NE_), jnp.float32),
            pltpu.SMEM((1,), jnp.float32),
        ],
    )(x, embed)

    ind_flat = ind3.reshape(n)

    g = _sc_gather(embed_t, ind_flat)

    quantize = g.reshape(input_lr.shape)
    diff = diff2.reshape(())
    embed_ind = ind_flat.reshape(input_lr.shape[:-1])
    return (quantize, diff, embed_ind, dist)


# speedup vs baseline: 3.7513x; 1.0774x over previous
import functools

import jax
import jax.numpy as jnp
from jax import lax
from jax.experimental import pallas as pl
from jax.experimental.pallas import tpu as pltpu
from jax.experimental.pallas import tpu_sc as plsc

DIM_ = 256
NE_ = 8192
TM_ = 512
N_ = 16384
SC_WORKERS_ = 32
SC_CHUNK_ = 256


W_ = 128


def _dist_body(x_ref, e_ref, d_ref, ind_ref, et_ref, diff_ref,
               eb_ref, e2_ref, acc_ref):
    i = pl.program_id(0)

    @pl.when(i == 0)
    def _prep():
        e = e_ref[...]
        et_ref[...] = e.T
        eb_ref[...] = (e * -2.0).astype(jnp.bfloat16)
        e2_ref[...] = jnp.sum(e * e, axis=0, keepdims=True)
        acc_ref[0] = 0.0

    xb = x_ref[...]
    s = jnp.sum(xb * xb, axis=1, keepdims=True)
    e2 = e2_ref[...]
    m2 = jnp.dot(xb.astype(jnp.bfloat16), eb_ref[...],
                 preferred_element_type=jnp.float32)
    nch = NE_ // W_
    RH = 64
    for h in range(TM_ // RH):
        rows = slice(h * RH, (h + 1) * RH)
        sh = s[rows]
        mh = m2[rows]
        rv = ri = None
        for j in range(nch):
            cols = slice(j * W_, (j + 1) * W_)
            dj = (sh + mh[:, cols]) + e2[:, cols]
            d_ref[rows, cols] = dj
            if j == 0:
                rv = dj
                ri = jnp.zeros((RH, W_), jnp.int32)
            else:
                upd = dj < rv
                rv = jnp.where(upd, dj, rv)
                ri = jnp.where(upd, jnp.int32(j), ri)
        minv = jnp.min(rv, axis=1, keepdims=True)
        lane = jax.lax.broadcasted_iota(jnp.int32, (RH, W_), 1)
        big = jnp.int32(2**30)
        idx_full = jnp.where(rv == minv, ri * W_ + lane, big)
        ind = jnp.min(idx_full, axis=1)
        ind_ref[0, 0, rows] = ind
        acc_ref[0] += jnp.sum(minv)

    @pl.when(i == pl.num_programs(0) - 1)
    def _fin():
        diff_ref[...] = jnp.full((1, 1), acc_ref[0] / (16384.0 * 256.0),
                                 jnp.float32)


def _sc_gather(embed_t, ind_flat):
    mesh = plsc.VectorSubcoreMesh(core_axis_name="c", subcore_axis_name="s")
    b_per_w = N_ // SC_WORKERS_

    @functools.partial(
        pl.kernel, mesh=mesh,
        out_type=jax.ShapeDtypeStruct((N_, DIM_), jnp.float32),
        scratch_types=[
            pltpu.VMEM((SC_CHUNK_,), jnp.int32),
            pltpu.VMEM((SC_CHUNK_, DIM_), jnp.float32),
            pltpu.SemaphoreType.DMA,
        ],
    )
    def k(table_hbm, idx_hbm, out_hbm, idx_v, rows_v, sem):
        wid = lax.axis_index("s") * 2 + lax.axis_index("c")
        base = wid * b_per_w

        @pl.loop(0, b_per_w, step=SC_CHUNK_)
        def _(off):
            pltpu.sync_copy(idx_hbm.at[pl.ds(base + off, SC_CHUNK_)], idx_v)
            pltpu.async_copy(table_hbm.at[idx_v], rows_v, sem).wait()
            pltpu.sync_copy(rows_v, out_hbm.at[pl.ds(base + off, SC_CHUNK_)])

    return k(embed_t, ind_flat)


def kernel(input_lr, embed):
    n = input_lr.shape[0] * input_lr.shape[1] * input_lr.shape[2]
    x = input_lr.reshape(n, DIM_)
    nrt = n // TM_

    dist, ind3, embed_t, diff2 = pl.pallas_call(
        _dist_body,
        grid=(nrt,),
        in_specs=[
            pl.BlockSpec((TM_, DIM_), lambda i: (i, 0)),
            pl.BlockSpec((DIM_, NE_), lambda i: (0, 0)),
        ],
        out_specs=[
            pl.BlockSpec((TM_, NE_), lambda i: (i, 0)),
            pl.BlockSpec((1, 1, TM_), lambda i: (i, 0, 0)),
            pl.BlockSpec((NE_, DIM_), lambda i: (0, 0)),
            pl.BlockSpec((1, 1), lambda i: (0, 0)),
        ],
        out_shape=[
            jax.ShapeDtypeStruct((n, NE_), jnp.float32),
            jax.ShapeDtypeStruct((nrt, 1, TM_), jnp.int32),
            jax.ShapeDtypeStruct((NE_, DIM_), jnp.float32),
            jax.ShapeDtypeStruct((1, 1), jnp.float32),
        ],
        scratch_shapes=[
            pltpu.VMEM((DIM_, NE_), jnp.bfloat16),
            pltpu.VMEM((1, NE_), jnp.float32),
            pltpu.SMEM((1,), jnp.float32),
        ],
    )(x, embed)

    ind_flat = ind3.reshape(n)

    g = _sc_gather(embed_t, ind_flat)

    quantize = g.reshape(input_lr.shape)
    diff = diff2.reshape(())
    embed_ind = ind_flat.reshape(input_lr.shape[:-1])
    return (quantize, diff, embed_ind, dist)


# speedup vs baseline: 3.7513x; 1.0000x over previous
import functools

import jax
import jax.numpy as jnp
from jax import lax
from jax.experimental import pallas as pl
from jax.experimental.pallas import tpu as pltpu
from jax.experimental.pallas import tpu_sc as plsc

DIM_ = 256
NE_ = 8192
TM_ = 512
N_ = 16384
SC_WORKERS_ = 32
SC_CHUNK_ = 128


W_ = 128


def _dist_body(x_ref, e_ref, d_ref, ind_ref, et_ref, diff_ref,
               eb_ref, e2_ref, acc_ref):
    i = pl.program_id(0)

    @pl.when(i == 0)
    def _prep():
        e = e_ref[...]
        et_ref[...] = e.T
        eb_ref[...] = (e * -2.0).astype(jnp.bfloat16)
        e2_ref[...] = jnp.sum(e * e, axis=0, keepdims=True)
        acc_ref[0] = 0.0

    xb = x_ref[...]
    s = jnp.sum(xb * xb, axis=1, keepdims=True)
    e2 = e2_ref[...]
    m2 = jnp.dot(xb.astype(jnp.bfloat16), eb_ref[...],
                 preferred_element_type=jnp.float32)
    nch = NE_ // W_
    RH = 64
    for h in range(TM_ // RH):
        rows = slice(h * RH, (h + 1) * RH)
        sh = s[rows]
        mh = m2[rows]
        rv = ri = None
        for j in range(nch):
            cols = slice(j * W_, (j + 1) * W_)
            dj = (sh + mh[:, cols]) + e2[:, cols]
            d_ref[rows, cols] = dj
            if j == 0:
                rv = dj
                ri = jnp.zeros((RH, W_), jnp.int32)
            else:
                upd = dj < rv
                rv = jnp.where(upd, dj, rv)
                ri = jnp.where(upd, jnp.int32(j), ri)
        minv = jnp.min(rv, axis=1, keepdims=True)
        lane = jax.lax.broadcasted_iota(jnp.int32, (RH, W_), 1)
        big = jnp.int32(2**30)
        idx_full = jnp.where(rv == minv, ri * W_ + lane, big)
        ind = jnp.min(idx_full, axis=1)
        ind_ref[0, 0, rows] = ind
        acc_ref[0] += jnp.sum(minv)

    @pl.when(i == pl.num_programs(0) - 1)
    def _fin():
        diff_ref[...] = jnp.full((1, 1), acc_ref[0] / (16384.0 * 256.0),
                                 jnp.float32)


def _sc_gather(embed_t, ind_flat):
    mesh = plsc.VectorSubcoreMesh(core_axis_name="c", subcore_axis_name="s")
    b_per_w = N_ // SC_WORKERS_
    nch = b_per_w // SC_CHUNK_

    @functools.partial(
        pl.kernel, mesh=mesh,
        out_type=jax.ShapeDtypeStruct((N_, DIM_), jnp.float32),
        scratch_types=[
            pltpu.VMEM((2, SC_CHUNK_), jnp.int32),
            pltpu.VMEM((2, SC_CHUNK_, DIM_), jnp.float32),
            pltpu.SemaphoreType.DMA,
            pltpu.SemaphoreType.DMA,
        ],
    )
    def k(table_hbm, idx_hbm, out_hbm, idx_v, rows_v, sem0, sem1):
        wid = lax.axis_index("s") * 2 + lax.axis_index("c")
        base = wid * b_per_w
        sems = (sem0, sem1)
        cps = [None, None]
        for c in range(2):
            pltpu.sync_copy(idx_hbm.at[pl.ds(base + c * SC_CHUNK_, SC_CHUNK_)],
                            idx_v.at[c])
            cps[c] = pltpu.async_copy(table_hbm.at[idx_v.at[c]], rows_v.at[c],
                                      sems[c])
        for c in range(nch):
            b = c % 2
            cps[b].wait()
            pltpu.sync_copy(rows_v.at[b],
                            out_hbm.at[pl.ds(base + c * SC_CHUNK_, SC_CHUNK_)])
            if c + 2 < nch:
                off = base + (c + 2) * SC_CHUNK_
                pltpu.sync_copy(idx_hbm.at[pl.ds(off, SC_CHUNK_)], idx_v.at[b])
                cps[b] = pltpu.async_copy(table_hbm.at[idx_v.at[b]],
                                          rows_v.at[b], sems[b])

    return k(embed_t, ind_flat)


def kernel(input_lr, embed):
    n = input_lr.shape[0] * input_lr.shape[1] * input_lr.shape[2]
    x = input_lr.reshape(n, DIM_)
    nrt = n // TM_

    dist, ind3, embed_t, diff2 = pl.pallas_call(
        _dist_body,
        grid=(nrt,),
        in_specs=[
            pl.BlockSpec((TM_, DIM_), lambda i: (i, 0)),
            pl.BlockSpec((DIM_, NE_), lambda i: (0, 0)),
        ],
        out_specs=[
            pl.BlockSpec((TM_, NE_), lambda i: (i, 0)),
            pl.BlockSpec((1, 1, TM_), lambda i: (i, 0, 0)),
            pl.BlockSpec((NE_, DIM_), lambda i: (0, 0)),
            pl.BlockSpec((1, 1), lambda i: (0, 0)),
        ],
        out_shape=[
            jax.ShapeDtypeStruct((n, NE_), jnp.float32),
            jax.ShapeDtypeStruct((nrt, 1, TM_), jnp.int32),
            jax.ShapeDtypeStruct((NE_, DIM_), jnp.float32),
            jax.ShapeDtypeStruct((1, 1), jnp.float32),
        ],
        scratch_shapes=[
            pltpu.VMEM((DIM_, NE_), jnp.bfloat16),
            pltpu.VMEM((1, NE_), jnp.float32),
            pltpu.SMEM((1,), jnp.float32),
        ],
    )(x, embed)

    ind_flat = ind3.reshape(n)

    g = _sc_gather(embed_t, ind_flat)

    quantize = g.reshape(input_lr.shape)
    diff = diff2.reshape(())
    embed_ind = ind_flat.reshape(input_lr.shape[:-1])
    return (quantize, diff, embed_ind, dist)
